# baseline (speedup 1.0000x reference)
import functools

import jax
import jax.numpy as jnp
from jax.experimental import pallas as pl
from jax.experimental.pallas import tpu as pltpu

B, N, C = 16, 2048, 3
NPT, KNN = 512, 32
DIN = 19


def _fps_body(xt_ref, out_ref):
    x0 = xt_ref[:, 0, :]
    x1 = xt_ref[:, 1, :]
    x2 = xt_ref[:, 2, :]
    lane = jax.lax.broadcasted_iota(jnp.int32, (B, N), 1)

    def body(i, carry):
        distance, far = carry
        msk = lane == far
        c0 = jnp.sum(jnp.where(msk, x0, 0.0), axis=1, keepdims=True)
        c1 = jnp.sum(jnp.where(msk, x1, 0.0), axis=1, keepdims=True)
        c2 = jnp.sum(jnp.where(msk, x2, 0.0), axis=1, keepdims=True)
        cvec = jnp.concatenate([c0, c1, c2], axis=1).reshape(B, 1, 3)
        out_ref[:, pl.ds(i, 1), :] = cvec
        d = (x0 - c0) ** 2 + (x1 - c1) ** 2 + (x2 - c2) ** 2
        distance = jnp.minimum(distance, d)
        m = jnp.max(distance, axis=1, keepdims=True)
        far = jnp.min(jnp.where(distance == m, lane, N), axis=1, keepdims=True)
        return distance, far

    dist0 = jnp.full((B, N), 1e10, jnp.float32)
    far0 = jnp.zeros((B, 1), jnp.int32)
    jax.lax.fori_loop(0, NPT, body, (dist0, far0))


def _mlp_body(x_ref, w1_ref, b1_ref, w2_ref, b2_ref, w3_ref, b3_ref, out_ref):
    x = x_ref[...]
    h = jnp.dot(x, w1_ref[...], preferred_element_type=jnp.float32) + b1_ref[...]
    h = jnp.maximum(h, 0.0)
    h = jnp.dot(h, w2_ref[...], preferred_element_type=jnp.float32) + b2_ref[...]
    h = jnp.maximum(h, 0.0)
    h = jnp.dot(h, w3_ref[...], preferred_element_type=jnp.float32) + b3_ref[...]
    out_ref[...] = jnp.max(h, axis=1, keepdims=True)


def _topk_body(x_ref, xt_ref, poolt_ref, out_ref, d_ref, acc_ref):
    RB = x_ref.shape[1]
    x = x_ref[0]
    xt = xt_ref[0]
    sqi = jnp.sum(x * x, axis=1, keepdims=True)
    sqj = jnp.sum(xt * xt, axis=0, keepdims=True)
    g = jnp.dot(x, xt, preferred_element_type=jnp.float32)
    d_ref[...] = sqi + sqj - 2.0 * g
    kcols = jax.lax.broadcasted_iota(jnp.int32, (RB, KNN), 1)
    poolt = poolt_ref[0]

    lane = jax.lax.broadcasted_iota(jnp.int32, (RB, N), 1)
    acc_ref[...] = jnp.zeros((RB, KNN), jnp.float32)

    def body(k, _):
        dmat = d_ref[...]
        m = jnp.min(dmat, axis=1, keepdims=True)
        ties = dmat == m
        jstar = jnp.min(jnp.where(ties, lane, N), axis=1, keepdims=True)
        oh = lane == jstar
        val = jnp.dot(jnp.where(oh, 1.0, 0.0), poolt,
                      preferred_element_type=jnp.float32)
        d_ref[...] = jnp.where(oh, jnp.inf, dmat)
        acc_ref[...] = jnp.where(kcols == k, val, acc_ref[...])
        return 0

    jax.lax.fori_loop(0, KNN, body, 0)
    out_ref[0] = acc_ref[...]


def kernel(xyz, features, W1, b1, W2, b2, W3, b3):
    xt = jnp.transpose(xyz, (0, 2, 1))

    new_xyz = pl.pallas_call(
        _fps_body,
        out_shape=jax.ShapeDtypeStruct((B, NPT, 3), jnp.float32),
    )(xt)

    pointfeat = jnp.concatenate([xyz, features], axis=-1).reshape(B * N, DIN)
    mlp_grid = 16
    rb_mlp = (B * N) // mlp_grid
    pooled = pl.pallas_call(
        _mlp_body,
        grid=(mlp_grid,),
        in_specs=[
            pl.BlockSpec((rb_mlp, DIN), lambda i: (i, 0)),
            pl.BlockSpec((DIN, 64), lambda i: (0, 0)),
            pl.BlockSpec((1, 64), lambda i: (0, 0)),
            pl.BlockSpec((64, 128), lambda i: (0, 0)),
            pl.BlockSpec((1, 128), lambda i: (0, 0)),
            pl.BlockSpec((128, 128), lambda i: (0, 0)),
            pl.BlockSpec((1, 128), lambda i: (0, 0)),
        ],
        out_specs=pl.BlockSpec((rb_mlp, 1), lambda i: (i, 0)),
        out_shape=jax.ShapeDtypeStruct((B * N, 1), jnp.float32),
    )(pointfeat, W1, b1.reshape(1, 64), W2, b2.reshape(1, 128), W3, b3.reshape(1, 128))
    pooled = pooled.reshape(B, N, 1)

    RB = 1024
    nrb = N // RB
    new_feat = pl.pallas_call(
        _topk_body,
        grid=(B, nrb),
        in_specs=[
            pl.BlockSpec((1, RB, 3), lambda b, r: (b, r, 0)),
            pl.BlockSpec((1, 3, N), lambda b, r: (b, 0, 0)),
            pl.BlockSpec((1, N, 1), lambda b, r: (b, 0, 0)),
        ],
        out_specs=pl.BlockSpec((1, RB, KNN), lambda b, r: (b, r, 0)),
        out_shape=jax.ShapeDtypeStruct((B, N, KNN), jnp.float32),
        scratch_shapes=[
            pltpu.VMEM((RB, N), jnp.float32),
            pltpu.VMEM((RB, KNN), jnp.float32),
        ],
    )(xyz, xt, pooled)

    return new_xyz, new_feat


# speedup vs baseline: 1.0308x; 1.0308x over previous
import functools

import jax
import jax.numpy as jnp
from jax.experimental import pallas as pl
from jax.experimental.pallas import tpu as pltpu

B, N, C = 16, 2048, 3
NPT, KNN = 512, 32
DIN = 19


def _fps_body(xt_ref, out_ref):
    x0 = xt_ref[:, 0, :]
    x1 = xt_ref[:, 1, :]
    x2 = xt_ref[:, 2, :]
    lane = jax.lax.broadcasted_iota(jnp.int32, (B, N), 1)

    def body(i, carry):
        distance, far = carry
        msk = lane == far
        c0 = jnp.sum(jnp.where(msk, x0, 0.0), axis=1, keepdims=True)
        c1 = jnp.sum(jnp.where(msk, x1, 0.0), axis=1, keepdims=True)
        c2 = jnp.sum(jnp.where(msk, x2, 0.0), axis=1, keepdims=True)
        cvec = jnp.concatenate([c0, c1, c2], axis=1).reshape(B, 1, 3)
        out_ref[:, pl.ds(i, 1), :] = cvec
        d = (x0 - c0) ** 2 + (x1 - c1) ** 2 + (x2 - c2) ** 2
        distance = jnp.minimum(distance, d)
        m = jnp.max(distance, axis=1, keepdims=True)
        far = jnp.min(jnp.where(distance == m, lane, N), axis=1, keepdims=True)
        return distance, far

    dist0 = jnp.full((B, N), 1e10, jnp.float32)
    far0 = jnp.zeros((B, 1), jnp.int32)
    jax.lax.fori_loop(0, NPT, body, (dist0, far0))


def _mlp_body(x_ref, w1_ref, b1_ref, w2_ref, b2_ref, w3_ref, b3_ref, out_ref):
    x = x_ref[...]
    h = jnp.dot(x, w1_ref[...], preferred_element_type=jnp.float32) + b1_ref[...]
    h = jnp.maximum(h, 0.0)
    h = jnp.dot(h, w2_ref[...], preferred_element_type=jnp.float32) + b2_ref[...]
    h = jnp.maximum(h, 0.0)
    h = jnp.dot(h, w3_ref[...], preferred_element_type=jnp.float32) + b3_ref[...]
    out_ref[...] = jnp.max(h, axis=1, keepdims=True)


def _topk_body(x_ref, xt_ref, pool_ref, out_ref, d_ref, acc_ref):
    RB = x_ref.shape[1]
    x = x_ref[0]
    xt = xt_ref[0]
    sqi = jnp.sum(x * x, axis=1, keepdims=True)
    sqj = jnp.sum(xt * xt, axis=0, keepdims=True)
    g = jnp.dot(x, xt, preferred_element_type=jnp.float32)
    d_ref[...] = sqi + sqj - 2.0 * g
    kcols = jax.lax.broadcasted_iota(jnp.int32, (RB, KNN), 1)
    pool = pool_ref[0]

    lane = jax.lax.broadcasted_iota(jnp.int32, (RB, N), 1)
    acc_ref[...] = jnp.zeros((RB, KNN), jnp.float32)

    def select(dmat):
        m = jnp.min(dmat, axis=1, keepdims=True)
        ties = dmat == m
        jstar = jnp.min(jnp.where(ties, lane, N), axis=1, keepdims=True)
        oh = lane == jstar
        val = jnp.sum(jnp.where(oh, pool, 0.0), axis=1, keepdims=True)
        return jnp.where(oh, jnp.inf, dmat), val

    def body(k2, _):
        dmat = d_ref[...]
        dmat, v1 = select(dmat)
        dmat, v2 = select(dmat)
        d_ref[...] = dmat
        acc_ref[...] = jnp.where(
            kcols == 2 * k2, v1,
            jnp.where(kcols == 2 * k2 + 1, v2, acc_ref[...]))
        return 0

    jax.lax.fori_loop(0, KNN // 2, body, 0)
    out_ref[0] = acc_ref[...]


def kernel(xyz, features, W1, b1, W2, b2, W3, b3):
    xt = jnp.transpose(xyz, (0, 2, 1))

    new_xyz = pl.pallas_call(
        _fps_body,
        out_shape=jax.ShapeDtypeStruct((B, NPT, 3), jnp.float32),
    )(xt)

    pointfeat = jnp.concatenate([xyz, features], axis=-1).reshape(B * N, DIN)
    mlp_grid = 16
    rb_mlp = (B * N) // mlp_grid
    pooled = pl.pallas_call(
        _mlp_body,
        grid=(mlp_grid,),
        in_specs=[
            pl.BlockSpec((rb_mlp, DIN), lambda i: (i, 0)),
            pl.BlockSpec((DIN, 64), lambda i: (0, 0)),
            pl.BlockSpec((1, 64), lambda i: (0, 0)),
            pl.BlockSpec((64, 128), lambda i: (0, 0)),
            pl.BlockSpec((1, 128), lambda i: (0, 0)),
            pl.BlockSpec((128, 128), lambda i: (0, 0)),
            pl.BlockSpec((1, 128), lambda i: (0, 0)),
        ],
        out_specs=pl.BlockSpec((rb_mlp, 1), lambda i: (i, 0)),
        out_shape=jax.ShapeDtypeStruct((B * N, 1), jnp.float32),
    )(pointfeat, W1, b1.reshape(1, 64), W2, b2.reshape(1, 128), W3, b3.reshape(1, 128))
    pooled = pooled.reshape(B, 1, N)

    RB = 1024
    nrb = N // RB
    new_feat = pl.pallas_call(
        _topk_body,
        grid=(B, nrb),
        in_specs=[
            pl.BlockSpec((1, RB, 3), lambda b, r: (b, r, 0)),
            pl.BlockSpec((1, 3, N), lambda b, r: (b, 0, 0)),
            pl.BlockSpec((1, 1, N), lambda b, r: (b, 0, 0)),
        ],
        out_specs=pl.BlockSpec((1, RB, KNN), lambda b, r: (b, r, 0)),
        out_shape=jax.ShapeDtypeStruct((B, N, KNN), jnp.float32),
        scratch_shapes=[
            pltpu.VMEM((RB, N), jnp.float32),
            pltpu.VMEM((RB, KNN), jnp.float32),
        ],
    )(xyz, xt, pooled)

    return new_xyz, new_feat


# speedup vs baseline: 1.0550x; 1.0235x over previous
import functools

import jax
import jax.numpy as jnp
from jax.experimental import pallas as pl
from jax.experimental.pallas import tpu as pltpu

B, N, C = 16, 2048, 3
NPT, KNN = 512, 32
DIN = 19


def _fps_body(xt_ref, out_ref):
    x0 = xt_ref[:, 0, :]
    x1 = xt_ref[:, 1, :]
    x2 = xt_ref[:, 2, :]
    lane = jax.lax.broadcasted_iota(jnp.int32, (B, N), 1)

    def body(i, carry):
        distance, far = carry
        msk = lane == far
        c0 = jnp.sum(jnp.where(msk, x0, 0.0), axis=1, keepdims=True)
        c1 = jnp.sum(jnp.where(msk, x1, 0.0), axis=1, keepdims=True)
        c2 = jnp.sum(jnp.where(msk, x2, 0.0), axis=1, keepdims=True)
        cvec = jnp.concatenate([c0, c1, c2], axis=1).reshape(B, 1, 3)
        out_ref[:, pl.ds(i, 1), :] = cvec
        d = (x0 - c0) ** 2 + (x1 - c1) ** 2 + (x2 - c2) ** 2
        distance = jnp.minimum(distance, d)
        m = jnp.max(distance, axis=1, keepdims=True)
        far = jnp.min(jnp.where(distance == m, lane, N), axis=1, keepdims=True)
        return distance, far

    dist0 = jnp.full((B, N), 1e10, jnp.float32)
    far0 = jnp.zeros((B, 1), jnp.int32)
    jax.lax.fori_loop(0, NPT, body, (dist0, far0))


def _mlp_body(x_ref, w1_ref, b1_ref, w2_ref, b2_ref, w3_ref, b3_ref, out_ref):
    x = x_ref[...]
    h = jnp.dot(x, w1_ref[...], preferred_element_type=jnp.float32) + b1_ref[...]
    h = jnp.maximum(h, 0.0)
    h = jnp.dot(h, w2_ref[...], preferred_element_type=jnp.float32) + b2_ref[...]
    h = jnp.maximum(h, 0.0)
    h = jnp.dot(h, w3_ref[...], preferred_element_type=jnp.float32) + b3_ref[...]
    out_ref[...] = jnp.max(h, axis=1, keepdims=True)


def _topk_body(x_ref, xt_ref, pool_ref, out_ref, d_ref, acc_ref):
    RB = x_ref.shape[1]
    x = x_ref[0]
    xt = xt_ref[0]
    sqi = jnp.sum(x * x, axis=1, keepdims=True)
    sqj = jnp.sum(xt * xt, axis=0, keepdims=True)
    g = jnp.dot(x, xt, preferred_element_type=jnp.float32)
    d_ref[...] = sqi + sqj - 2.0 * g
    kcols = jax.lax.broadcasted_iota(jnp.int32, (RB, KNN), 1)
    pool = pool_ref[0]

    lane = jax.lax.broadcasted_iota(jnp.int32, (RB, N), 1)
    acc_ref[...] = jnp.zeros((RB, KNN), jnp.float32)

    def body(k, _):
        dmat = d_ref[...]
        m = jnp.min(dmat, axis=1, keepdims=True)
        ties = dmat == m
        jstar = jnp.min(jnp.where(ties, lane, N), axis=1, keepdims=True)
        oh = lane == jstar
        val = jnp.sum(jnp.where(oh, pool, 0.0), axis=1, keepdims=True)
        d_ref[...] = jnp.where(oh, jnp.inf, dmat)
        acc_ref[...] = jnp.where(kcols == k, val, acc_ref[...])
        return 0

    jax.lax.fori_loop(0, KNN, body, 0)
    out_ref[0] = acc_ref[...]


def kernel(xyz, features, W1, b1, W2, b2, W3, b3):
    xt = jnp.transpose(xyz, (0, 2, 1))

    new_xyz = pl.pallas_call(
        _fps_body,
        out_shape=jax.ShapeDtypeStruct((B, NPT, 3), jnp.float32),
    )(xt)

    pointfeat = jnp.concatenate([xyz, features], axis=-1).reshape(B * N, DIN)
    mlp_grid = 16
    rb_mlp = (B * N) // mlp_grid
    pooled = pl.pallas_call(
        _mlp_body,
        grid=(mlp_grid,),
        in_specs=[
            pl.BlockSpec((rb_mlp, DIN), lambda i: (i, 0)),
            pl.BlockSpec((DIN, 64), lambda i: (0, 0)),
            pl.BlockSpec((1, 64), lambda i: (0, 0)),
            pl.BlockSpec((64, 128), lambda i: (0, 0)),
            pl.BlockSpec((1, 128), lambda i: (0, 0)),
            pl.BlockSpec((128, 128), lambda i: (0, 0)),
            pl.BlockSpec((1, 128), lambda i: (0, 0)),
        ],
        out_specs=pl.BlockSpec((rb_mlp, 1), lambda i: (i, 0)),
        out_shape=jax.ShapeDtypeStruct((B * N, 1), jnp.float32),
    )(pointfeat, W1, b1.reshape(1, 64), W2, b2.reshape(1, 128), W3, b3.reshape(1, 128))
    pooled = pooled.reshape(B, 1, N)

    RB = 1024
    nrb = N // RB
    new_feat = pl.pallas_call(
        _topk_body,
        grid=(B, nrb),
        in_specs=[
            pl.BlockSpec((1, RB, 3), lambda b, r: (b, r, 0)),
            pl.BlockSpec((1, 3, N), lambda b, r: (b, 0, 0)),
            pl.BlockSpec((1, 1, N), lambda b, r: (b, 0, 0)),
        ],
        out_specs=pl.BlockSpec((1, RB, KNN), lambda b, r: (b, r, 0)),
        out_shape=jax.ShapeDtypeStruct((B, N, KNN), jnp.float32),
        scratch_shapes=[
            pltpu.VMEM((RB, N), jnp.float32),
            pltpu.VMEM((RB, KNN), jnp.float32),
        ],
    )(xyz, xt, pooled)

    return new_xyz, new_feat
